# baseline (speedup 1.0000x reference)
import functools

import jax
import jax.numpy as jnp
from jax import lax
from jax.experimental import pallas as pl
from jax.experimental.pallas import tpu as pltpu
from jax.experimental.pallas import tpu_sc as plsc

B = 16384
F = 32
LINE = 128
RPL = LINE // F
NC = 2
NS = 16
NW = NC * NS
BPW = B // NW
CHUNK = 128
NCHUNK = BPW // CHUNK
NBUF = 2
L = 16


def _sc_body(users_hbm, movies_hbm, uemb_hbm, memb_hbm, ubias_hbm, mbias_hbm,
             out_hbm, uidx_v, midx_v, uline_v, mline_v, ubuf_v, mbuf_v,
             ubias_v, mbias_v, dot_v, gsem0, gsem1, bsem):
    gsems = (gsem0, gsem1)
    wid = lax.axis_index("s") * NC + lax.axis_index("c")
    base = wid * BPW
    lane = lax.iota(jnp.int32, L)

    for j in range(NCHUNK):
        off = base + j * CHUNK
        pltpu.sync_copy(users_hbm.at[pl.ds(off, CHUNK)], uidx_v.at[j])
        pltpu.sync_copy(movies_hbm.at[pl.ds(off, CHUNK)], midx_v.at[j])

    bias_copies = []
    for j in range(NCHUNK):
        dst = pl.ds(j * CHUNK, CHUNK)
        bias_copies.append(pltpu.async_copy(ubias_hbm.at[uidx_v.at[j]],
                                            ubias_v.at[dst], bsem))
        bias_copies.append(pltpu.async_copy(mbias_hbm.at[midx_v.at[j]],
                                            mbias_v.at[dst], bsem))

    for j in range(NCHUNK):
        for g in range(CHUNK // L):
            s = pl.ds(g * L, L)
            uline_v[j, s] = lax.shift_right_logical(uidx_v[j, s], RPL // 2)
            mline_v[j, s] = lax.shift_right_logical(midx_v[j, s], RPL // 2)

    def start_chunk(j):
        b = j % NBUF
        return (pltpu.async_copy(uemb_hbm.at[uline_v.at[j]], ubuf_v.at[b],
                                 gsems[b]),
                pltpu.async_copy(memb_hbm.at[mline_v.at[j]], mbuf_v.at[b],
                                 gsems[b]))

    inflight = [start_chunk(0)]

    def compute_chunk(j):
        b = j % NBUF
        for g in range(CHUNK // L):
            s = pl.ds(g * L, L)
            uidx = uidx_v[j, s]
            midx = midx_v[j, s]
            ucol = (uidx & (RPL - 1)) * F
            mcol = (midx & (RPL - 1)) * F
            acc = ubias_v[pl.ds(j * CHUNK + g * L, L)] + \
                mbias_v[pl.ds(j * CHUNK + g * L, L)]
            rows = g * L + lane
            for f in range(F):
                acc += (plsc.load_gather(ubuf_v.at[b], [rows, ucol + f]) *
                        plsc.load_gather(mbuf_v.at[b], [rows, mcol + f]))
            dot_v[pl.ds(j * CHUNK + g * L, L)] = 1.0 / (1.0 + jnp.exp(-acc))

    for c in bias_copies:
        c.wait()
    for j in range(NCHUNK):
        if j + 1 < NCHUNK:
            inflight.append(start_chunk(j + 1))
        for c in inflight[j]:
            c.wait()
        compute_chunk(j)

    pltpu.sync_copy(dot_v, out_hbm.at[pl.ds(base, BPW)])


@jax.jit
def _mf_sc(users, movies, uemb, memb, ubias1d, mbias1d):
    mesh = plsc.VectorSubcoreMesh(core_axis_name="c", subcore_axis_name="s")
    return pl.kernel(
        _sc_body,
        out_type=jax.ShapeDtypeStruct((B,), jnp.float32),
        mesh=mesh,
        compiler_params=pltpu.CompilerParams(needs_layout_passes=False),
        scratch_types=[
            pltpu.VMEM((NCHUNK, CHUNK), jnp.int32),
            pltpu.VMEM((NCHUNK, CHUNK), jnp.int32),
            pltpu.VMEM((NCHUNK, CHUNK), jnp.int32),
            pltpu.VMEM((NCHUNK, CHUNK), jnp.int32),
            pltpu.VMEM((NBUF, CHUNK, LINE), jnp.float32),
            pltpu.VMEM((NBUF, CHUNK, LINE), jnp.float32),
            pltpu.VMEM((BPW,), jnp.float32),
            pltpu.VMEM((BPW,), jnp.float32),
            pltpu.VMEM((BPW,), jnp.float32),
            pltpu.SemaphoreType.DMA,
            pltpu.SemaphoreType.DMA,
            pltpu.SemaphoreType.DMA,
        ],
    )(users, movies, uemb, memb, ubias1d, mbias1d)


def kernel(users, movies, user_embedding, movie_embedding, user_bias,
           movie_bias):
    return _mf_sc(users, movies,
                  user_embedding.reshape(-1, LINE),
                  movie_embedding.reshape(-1, LINE),
                  user_bias.reshape(-1), movie_bias.reshape(-1))
